# baseline (speedup 1.0000x reference)
import functools

import jax
import jax.numpy as jnp
from jax import lax
from jax.experimental import pallas as pl
from jax.experimental.pallas import tpu as pltpu
from jax.experimental.pallas import tpu_sc as plsc

_NC = 2
_NS = 16
_NW = _NC * _NS

_N = 4096
_ND = 4
_NCONT = 32
_DIM = 512
_NROW = _ND + _NCONT
_PW = _N // _NW
_PP = 2
_STEPS = _PW // _PP


def _sc_body(idx_hbm, cont_hbm, dtab_hbm, ctab_hbm, out_hbm,
             idx_v, cont_v, ctab_v, gbuf, obuf, gsem):
    wid = lax.axis_index("s") * _NC + lax.axis_index("c")
    p0 = wid * _PW

    pltpu.sync_copy(idx_hbm.at[pl.ds(p0 * _ND, _PW * _ND)], idx_v)
    pltpu.sync_copy(cont_hbm.at[pl.ds(p0, _PW)], cont_v)
    pltpu.sync_copy(ctab_hbm, ctab_v)

    def step(i, carry):
        off = pl.multiple_of(i * (_PP * _ND), 8)
        pltpu.async_copy(dtab_hbm.at[idx_v.at[pl.ds(off, _PP * _ND)]],
                         gbuf, gsem).wait()

        def cpk(k, c):
            ks = pl.ds(k * 16, 16)
            for pp in range(_PP):
                for r in range(_ND):
                    obuf[pp, r, ks] = gbuf[pp * _ND + r, ks]
            return c
        lax.fori_loop(0, _DIM // 16, cpk, 0)

        for pp in range(_PP):
            pos = i * _PP + pp
            cv0 = cont_v[pos, pl.ds(0, 16)]
            cv1 = cont_v[pos, pl.ds(16, 16)]
            cs = [cv0[j] for j in range(16)] + [cv1[j] for j in range(16)]

            def ck(k, c2):
                ks = pl.ds(k * 16, 16)
                for j in range(_NCONT):
                    obuf[pp, _ND + j, ks] = cs[j] * ctab_v[j, ks]
                return c2
            lax.fori_loop(0, _DIM // 16, ck, 0)

        p = p0 + i * _PP
        pltpu.sync_copy(obuf, out_hbm.at[p // (_N // 2),
                                         pl.ds(p % (_N // 2), _PP)])
        return carry

    lax.fori_loop(0, _STEPS, step, 0)


@jax.jit
def _sc_call(flat_idx, cont, disc_table, cont_table):
    mesh = plsc.VectorSubcoreMesh(core_axis_name="c", subcore_axis_name="s")
    f = functools.partial(
        pl.kernel, _sc_body, mesh=mesh,
        out_type=jax.ShapeDtypeStruct((2, _N // 2, _NROW, _DIM), jnp.float32),
        scratch_types=[
            pltpu.VMEM((_PW * _ND,), jnp.int32),
            pltpu.VMEM((_PW, _NCONT), jnp.float32),
            pltpu.VMEM((_NCONT, _DIM), jnp.float32),
            pltpu.VMEM((_PP * _ND, _DIM), jnp.float32),
            pltpu.VMEM((_PP, _NROW, _DIM), jnp.float32),
            pltpu.SemaphoreType.DMA,
        ],
    )()
    return f(flat_idx, cont, disc_table, cont_table)


def kernel(discrete_actions, continuous_actions, disc_table, cont_table, offsets):
    b, s, n_disc = discrete_actions.shape
    n_cont = continuous_actions.shape[-1]
    dim = disc_table.shape[-1]
    n = b * s
    flat_idx = (discrete_actions + offsets[None, None, :]).reshape(n * n_disc)
    cont = continuous_actions.reshape(n, n_cont)
    out = _sc_call(flat_idx, cont, disc_table, cont_table)
    return out.reshape(b, s, n_disc + n_cont, dim)


# speedup vs baseline: 1.4256x; 1.4256x over previous
import functools

import jax
import jax.numpy as jnp
from jax import lax
from jax.experimental import pallas as pl
from jax.experimental.pallas import tpu as pltpu
from jax.experimental.pallas import tpu_sc as plsc

_NC = 2
_NS = 16
_NW = _NC * _NS

_N = 4096
_S = 2048
_ND = 4
_NCONT = 32
_DIM = 512
_NROW = _ND + _NCONT
_HEAD = 8
_TAIL = _NROW - _HEAD
_PW = _N // _NW
_PP = 2
_STEPS = _PW // _PP


def _sc_body(idx_hbm, cont_hbm, dtab_hbm, ctab_hbm, out_hbm,
             idx_v, cont_v, ctab_v, gbuf, abuf, cbuf, gsem, osem0, osem1):
    wid = lax.axis_index("s") * _NC + lax.axis_index("c")
    p0 = wid * _PW
    bsel = p0 // _S
    sbase = p0 % _S
    osem = (osem0, osem1)

    pltpu.sync_copy(idx_hbm.at[pl.ds(p0 * _ND, _PW * _ND)], idx_v)
    pltpu.sync_copy(cont_hbm.at[pl.ds(p0, _PW)], cont_v)
    pltpu.sync_copy(ctab_hbm, ctab_v)

    def do_step(s, nb):
        off = pl.multiple_of(s * (_PP * _ND), 8)
        gh = pltpu.async_copy(dtab_hbm.at[idx_v.at[pl.ds(off, _PP * _ND)]],
                              gbuf.at[nb], gsem)

        scal = []
        for pp in range(_PP):
            pos = s * _PP + pp
            cv0 = cont_v[pos, pl.ds(0, 16)]
            cv1 = cont_v[pos, pl.ds(16, 16)]
            scal.append([cv0[j] for j in range(16)] + [cv1[j] for j in range(16)])

        def ck(k, c):
            ks = pl.ds(k * 16, 16)
            for j in range(_NCONT):
                row = ctab_v[j, ks]
                for pp in range(_PP):
                    if j < _HEAD - _ND:
                        abuf[nb, pp, _ND + j, ks] = scal[pp][j] * row
                    else:
                        cbuf[nb, pp, j - (_HEAD - _ND), ks] = scal[pp][j] * row
            return c
        lax.fori_loop(0, _DIM // 16, ck, 0)

        gh.wait()

        def cpk(k, c):
            ks = pl.ds(k * 16, 16)
            for pp in range(_PP):
                for r in range(_ND):
                    abuf[nb, pp, r, ks] = gbuf[nb, pp * _ND + r, ks]
            return c
        lax.fori_loop(0, _DIM // 16, cpk, 0)

        spos = sbase + s * _PP
        for pp in range(_PP):
            pltpu.async_copy(abuf.at[nb, pp],
                             out_hbm.at[bsel, spos + pp, pl.ds(0, _HEAD)],
                             osem[nb])
            pltpu.async_copy(cbuf.at[nb, pp],
                             out_hbm.at[bsel, spos + pp, pl.ds(_HEAD, _TAIL)],
                             osem[nb])

    def drain(nb):
        pltpu.make_async_copy(out_hbm.at[0, pl.ds(0, _PP), pl.ds(0, _HEAD)],
                              abuf.at[nb], osem[nb]).wait()
        pltpu.make_async_copy(out_hbm.at[0, pl.ds(0, _PP), pl.ds(_HEAD, _TAIL)],
                              cbuf.at[nb], osem[nb]).wait()

    do_step(0, 0)
    do_step(1, 1)

    def outer(s2, c):
        for nb in range(2):
            drain(nb)
            do_step(s2 * 2 + nb, nb)
        return c
    lax.fori_loop(1, _STEPS // 2, outer, 0)
    drain(0)
    drain(1)


@jax.jit
def _sc_call(flat_idx, cont, disc_table, cont_table):
    mesh = plsc.VectorSubcoreMesh(core_axis_name="c", subcore_axis_name="s")
    f = functools.partial(
        pl.kernel, _sc_body, mesh=mesh,
        out_type=jax.ShapeDtypeStruct((_N // _S, _S, _NROW, _DIM), jnp.float32),
        scratch_types=[
            pltpu.VMEM((_PW * _ND,), jnp.int32),
            pltpu.VMEM((_PW, _NCONT), jnp.float32),
            pltpu.VMEM((_NCONT, _DIM), jnp.float32),
            pltpu.VMEM((2, _PP * _ND, _DIM), jnp.float32),
            pltpu.VMEM((2, _PP, _HEAD, _DIM), jnp.float32),
            pltpu.VMEM((2, _PP, _TAIL, _DIM), jnp.float32),
            pltpu.SemaphoreType.DMA,
            pltpu.SemaphoreType.DMA,
            pltpu.SemaphoreType.DMA,
        ],
    )()
    return f(flat_idx, cont, disc_table, cont_table)


def kernel(discrete_actions, continuous_actions, disc_table, cont_table, offsets):
    b, s, n_disc = discrete_actions.shape
    n_cont = continuous_actions.shape[-1]
    dim = disc_table.shape[-1]
    n = b * s
    flat_idx = (discrete_actions + offsets[None, None, :]).reshape(n * n_disc)
    cont = continuous_actions.reshape(n, n_cont)
    out = _sc_call(flat_idx, cont, disc_table, cont_table)
    return out.reshape(b, s, n_disc + n_cont, dim)
